# baseline (speedup 1.0000x reference)
import functools
import math

import jax
import jax.numpy as jnp
from jax import lax
from jax.experimental import pallas as pl

E = 8
TOP_K = 2
D = 768
H = 256
EPS_LN = 1e-5

BN = 256


def _moe_body(x_ref, wr_ref, br_ref, w1_ref, b1_ref, g1_ref, beta1_ref,
              w2cat_ref, b2_ref, out_ref):
    xb = x_ref[...]

    logits = jnp.dot(xb, wr_ref[...], preferred_element_type=jnp.float32)
    logits = logits + br_ref[...]
    m = jnp.max(logits, axis=-1, keepdims=True)
    p = jnp.exp(logits - m)
    p = p / jnp.sum(p, axis=-1, keepdims=True)

    iota_e = lax.broadcasted_iota(jnp.int32, (BN, E), 1)
    m1 = jnp.max(p, axis=-1, keepdims=True)
    idx1 = jnp.min(jnp.where(p == m1, iota_e, E), axis=-1, keepdims=True)
    mask1 = iota_e == idx1
    p_rest = jnp.where(mask1, -jnp.inf, p)
    m2 = jnp.max(p_rest, axis=-1, keepdims=True)
    idx2 = jnp.min(jnp.where(p_rest == m2, iota_e, E), axis=-1, keepdims=True)
    mask2 = iota_e == idx2
    denom = jnp.maximum(m1 + m2, 1e-9)
    w = jnp.where(mask1 | mask2, p, 0.0) / denom

    b1 = b1_ref[...]
    g1 = g1_ref[...]
    beta1 = beta1_ref[...]

    inv_sqrt2 = 1.0 / math.sqrt(2.0)
    w_half = 0.5 * w
    iota_h2 = lax.broadcasted_iota(jnp.int32, (2 * H, 2), 0)
    iota_c2 = lax.broadcasted_iota(jnp.int32, (2 * H, 2), 1)
    sel = jnp.where((iota_h2 // H) == iota_c2, 1.0, 0.0)
    chunks = []
    for e in range(E):
        h = jnp.dot(xb, w1_ref[e], preferred_element_type=jnp.float32)
        h = h + b1[e][None, :]
        hh = jnp.concatenate([h, h * h], axis=-1)
        s12 = jnp.dot(hh, sel, preferred_element_type=jnp.float32)
        mu = s12[:, 0][:, None] * (1.0 / H)
        var = s12[:, 1][:, None] * (1.0 / H) - mu * mu
        hn = (h - mu) * lax.rsqrt(var + EPS_LN)
        hn = hn * g1[e][None, :] + beta1[e][None, :]
        t = w_half[:, e][:, None] * hn
        chunks.append(t + t * lax.erf(hn * inv_sqrt2))
    a_all = jnp.concatenate(chunks, axis=-1)
    acc = jnp.dot(a_all, w2cat_ref[...], preferred_element_type=jnp.float32)
    acc = acc + jnp.dot(w, b2_ref[...], preferred_element_type=jnp.float32)
    out_ref[...] = acc


@functools.partial(jax.jit, static_argnames=("interpret",))
def kernel(x, Wr, br, W1, b1, g1, beta1, W2, b2, interpret=False):
    orig_shape = x.shape
    x2 = x.reshape(-1, x.shape[-1])
    n = x2.shape[0]
    grid = (n // BN,)
    out = pl.pallas_call(
        _moe_body,
        grid=grid,
        in_specs=[
            pl.BlockSpec((BN, D), lambda i: (i, 0)),
            pl.BlockSpec((D, E), lambda i: (0, 0)),
            pl.BlockSpec((1, E), lambda i: (0, 0)),
            pl.BlockSpec((E, D, H), lambda i: (0, 0, 0)),
            pl.BlockSpec((E, H), lambda i: (0, 0)),
            pl.BlockSpec((E, H), lambda i: (0, 0)),
            pl.BlockSpec((E, H), lambda i: (0, 0)),
            pl.BlockSpec((E * H, D), lambda i: (0, 0)),
            pl.BlockSpec((E, D), lambda i: (0, 0)),
        ],
        out_specs=pl.BlockSpec((BN, D), lambda i: (i, 0)),
        out_shape=jax.ShapeDtypeStruct((n, D), jnp.float32),
        interpret=interpret,
    )(x2, Wr, br.reshape(1, E), W1, b1, g1, beta1,
      W2.reshape(E * H, D), b2)
    return out.reshape(orig_shape)


# speedup vs baseline: 1.2542x; 1.2542x over previous
import functools
import math

import jax
import jax.numpy as jnp
from jax import lax
from jax.experimental import pallas as pl

E = 8
TOP_K = 2
D = 768
H = 256
EPS_LN = 1e-5

BN = 512


def _moe_body(x_ref, wr_ref, br_ref, w1_ref, b1_ref, g1_ref, beta1_ref,
              w2cat_ref, b2_ref, out_ref):
    xb = x_ref[...]

    logits = jnp.dot(xb, wr_ref[...], preferred_element_type=jnp.float32)
    logits = logits + br_ref[...]
    m = jnp.max(logits, axis=-1, keepdims=True)
    p = jnp.exp(logits - m)
    p = p / jnp.sum(p, axis=-1, keepdims=True)

    iota_e = lax.broadcasted_iota(jnp.int32, (BN, E), 1)
    m1 = jnp.max(p, axis=-1, keepdims=True)
    idx1 = jnp.min(jnp.where(p == m1, iota_e, E), axis=-1, keepdims=True)
    mask1 = iota_e == idx1
    p_rest = jnp.where(mask1, -jnp.inf, p)
    m2 = jnp.max(p_rest, axis=-1, keepdims=True)
    idx2 = jnp.min(jnp.where(p_rest == m2, iota_e, E), axis=-1, keepdims=True)
    mask2 = iota_e == idx2
    denom = jnp.maximum(m1 + m2, 1e-9)
    w = jnp.where(mask1 | mask2, p, 0.0) / denom

    b1 = b1_ref[...]
    g1 = g1_ref[...]
    beta1 = beta1_ref[...]

    inv_sqrt2 = 1.0 / math.sqrt(2.0)
    w_half = 0.5 * w
    chunks = []
    for e in range(E):
        h = jnp.dot(xb, w1_ref[e], preferred_element_type=jnp.float32)
        h = h + b1[e][None, :]
        s1 = jnp.sum(h, axis=-1, keepdims=True)
        s2 = jnp.sum(h * h, axis=-1, keepdims=True)
        mu = s1 * (1.0 / H)
        var = s2 * (1.0 / H) - mu * mu
        hn = (h - mu) * lax.rsqrt(var + EPS_LN)
        hn = hn * g1[e][None, :] + beta1[e][None, :]
        t = w_half[:, e][:, None] * hn
        chunks.append(t + t * lax.erf(hn * inv_sqrt2))
    a_all = jnp.concatenate(chunks, axis=-1)
    acc = jnp.dot(a_all, w2cat_ref[...], preferred_element_type=jnp.float32)
    acc = acc + jnp.dot(w, b2_ref[...], preferred_element_type=jnp.float32)
    out_ref[...] = acc


@functools.partial(jax.jit, static_argnames=("interpret",))
def kernel(x, Wr, br, W1, b1, g1, beta1, W2, b2, interpret=False):
    orig_shape = x.shape
    x2 = x.reshape(-1, x.shape[-1])
    n = x2.shape[0]
    grid = (n // BN,)
    out = pl.pallas_call(
        _moe_body,
        grid=grid,
        in_specs=[
            pl.BlockSpec((BN, D), lambda i: (i, 0)),
            pl.BlockSpec((D, E), lambda i: (0, 0)),
            pl.BlockSpec((1, E), lambda i: (0, 0)),
            pl.BlockSpec((E, D, H), lambda i: (0, 0, 0)),
            pl.BlockSpec((E, H), lambda i: (0, 0)),
            pl.BlockSpec((E, H), lambda i: (0, 0)),
            pl.BlockSpec((E, H), lambda i: (0, 0)),
            pl.BlockSpec((E * H, D), lambda i: (0, 0)),
            pl.BlockSpec((E, D), lambda i: (0, 0)),
        ],
        out_specs=pl.BlockSpec((BN, D), lambda i: (i, 0)),
        out_shape=jax.ShapeDtypeStruct((n, D), jnp.float32),
        interpret=interpret,
    )(x2, Wr, br.reshape(1, E), W1, b1, g1, beta1,
      W2.reshape(E * H, D), b2)
    return out.reshape(orig_shape)
